# baseline (speedup 1.0000x reference)
import jax
import jax.numpy as jnp
from jax import lax
from jax.experimental import pallas as pl
from jax.experimental.pallas import tpu as pltpu
from jax.experimental.pallas import tpu_sc as plsc

_N = 10000
_NP = 10240
_E = 320000
_F = 128
_C = 40
_NT = 32
_CH = 125
_NSC = 5
_SCH = 16
_ECT = 10000
_RPT = _NP // 16
_RB = 128


def _sc_mesh():
    return plsc.VectorSubcoreMesh(core_axis_name="c", subcore_axis_name="s")


def _make_prop(D):
    def body(g_hbm, row_hbm, col_hbm, z_hbm, out_hbm,
             acc, rowv, colv, buf0, buf1, sem0, sem1):
        cid = lax.axis_index("c")
        sid = lax.axis_index("s")
        wid = sid * 2 + cid

        pltpu.sync_copy(z_hbm, buf0)
        base = sid * _RPT
        for k in range(_RPT // 80):
            pltpu.sync_copy(buf0.at[pl.ds(0, 80), :],
                            acc.at[pl.ds(base + k * 80, 80), :])
        plsc.subcore_barrier()

        for sc in range(_NSC):
            ridma = pltpu.async_copy(
                row_hbm.at[wid, pl.ds(sc * _SCH, _SCH), :], rowv, sem0)
            cidma = pltpu.async_copy(
                col_hbm.at[wid, pl.ds(sc * _SCH, _SCH), :], colv, sem1)
            ridma.wait()
            cidma.wait()

            pltpu.async_copy(g_hbm.at[rowv.at[0]], buf0, sem0)
            pltpu.async_copy(g_hbm.at[rowv.at[1]], buf1, sem1)

            def step(i, carry):
                a = 2 * i
                b = a + 1
                pltpu.make_async_copy(g_hbm.at[rowv.at[a]], buf0, sem0).wait()
                pltpu.sync_copy(buf0, acc.at[colv.at[a]], add=True)
                pltpu.async_copy(g_hbm.at[rowv.at[a + 2]], buf0, sem0)
                pltpu.make_async_copy(g_hbm.at[rowv.at[b]], buf1, sem1).wait()
                pltpu.sync_copy(buf1, acc.at[colv.at[b]], add=True)
                pltpu.async_copy(g_hbm.at[rowv.at[b + 2]], buf1, sem1)
                return carry

            lax.fori_loop(0, _SCH // 2 - 1, step, 0)
            pltpu.make_async_copy(g_hbm.at[rowv.at[_SCH - 2]], buf0,
                                  sem0).wait()
            pltpu.sync_copy(buf0, acc.at[colv.at[_SCH - 2]], add=True)
            pltpu.make_async_copy(g_hbm.at[rowv.at[_SCH - 1]], buf1,
                                  sem1).wait()
            pltpu.sync_copy(buf1, acc.at[colv.at[_SCH - 1]], add=True)

        plsc.subcore_barrier()
        for k in range(_RPT // 80):
            r0 = base + k * 80
            pltpu.sync_copy(acc.at[pl.ds(r0, 80), :], buf0.at[pl.ds(0, 80), :])
            pltpu.sync_copy(buf0.at[pl.ds(0, 80), :],
                            out_hbm.at[cid, pl.ds(r0, 80), :])

    return pl.kernel(
        body,
        out_type=jax.ShapeDtypeStruct((2, _NP, D), jnp.float32),
        mesh=_sc_mesh(),
        compiler_params=pltpu.CompilerParams(
            disable_bounds_checks=True, disable_semaphore_checks=True),
        scratch_types=[
            pltpu.VMEM_SHARED((_NP, D), jnp.float32),
            pltpu.VMEM((_SCH, _CH), jnp.int32),
            pltpu.VMEM((_SCH, _CH), jnp.int32),
            pltpu.VMEM((_CH, D), jnp.float32),
            pltpu.VMEM((_CH, D), jnp.float32),
            pltpu.SemaphoreType.DMA,
            pltpu.SemaphoreType.DMA,
        ],
    )


_DR = _NP // _F


def _deg_body(row_hbm, rix_hbm, out_hbm, acc2, degv, rowv, rix, sem0):
    cid = lax.axis_index("c")
    sid = lax.axis_index("s")
    wid = sid * 2 + cid

    ridma = pltpu.async_copy(row_hbm.at[wid], rowv, sem0)
    pltpu.sync_copy(rix_hbm, rix)
    z16 = jnp.zeros((16,), jnp.float32)

    def zstep(r, carry):
        for j in range(_F // 16):
            degv[r, pl.ds(j * 16, 16)] = z16
        return carry

    lax.fori_loop(0, _DR, zstep, 0)
    @pl.when(sid < _DR // 8)
    def _():
        pltpu.sync_copy(degv.at[pl.ds(sid * 8, 8), :],
                        acc2.at[pl.ds(sid * 8, 8), :])

    ridma.wait()
    plsc.subcore_barrier()

    ones16 = jnp.ones((16,), jnp.float32)

    def step(c, carry):
        idx = rowv[pl.ds(c * 16, 16)]
        plsc.addupdate_scatter(degv, [idx >> 7, idx & 127], ones16)
        return carry

    lax.fori_loop(0, _ECT // 16, step, 0)
    pltpu.sync_copy(degv, acc2.at[rix], add=True)
    plsc.subcore_barrier()

    @pl.when(sid < _DR // 8)
    def _():
        pltpu.sync_copy(acc2.at[pl.ds(sid * 8, 8), :],
                        degv.at[pl.ds(0, 8), :])
        pltpu.sync_copy(degv.at[pl.ds(0, 8), :],
                        out_hbm.at[cid, pl.ds(sid * 8, 8), :])


def _make_deg():
    return pl.kernel(
        _deg_body,
        out_type=jax.ShapeDtypeStruct((2, _DR, _F), jnp.float32),
        mesh=_sc_mesh(),
        compiler_params=pltpu.CompilerParams(
            needs_layout_passes=False,
            disable_bounds_checks=True, disable_semaphore_checks=True),
        scratch_types=[
            pltpu.VMEM_SHARED((_DR, _F), jnp.float32),
            pltpu.VMEM((_DR, _F), jnp.float32),
            pltpu.VMEM((_ECT,), jnp.int32),
            pltpu.VMEM((_DR,), jnp.int32),
            pltpu.SemaphoreType.DMA,
        ],
    )


_prop128 = _make_prop(_F)
_deg = _make_deg()


_BLK = 2000
_GRID = _N // _BLK


def _full(shape):
    nd = len(shape)
    return pl.BlockSpec(shape, lambda i, _nd=nd: (0,) * _nd)


def _rows(shape):
    if len(shape) == 3:
        return pl.BlockSpec(shape, lambda i: (0, i, 0))
    return pl.BlockSpec(shape, lambda i: (i, 0))


def _pre_body(deg_ref, x_ref, s_ref, sh_ref):
    d = deg_ref[0] + deg_ref[1]
    s = jnp.where(d > 0.0, lax.rsqrt(d), 0.0)
    s_ref[...] = s
    sh_ref[...] = x_ref[...] * s


_pre = pl.pallas_call(
    _pre_body,
    grid=(_GRID,),
    in_specs=[_rows((2, _BLK, 1)), _rows((_BLK, _F))],
    out_specs=[_rows((_BLK, 1)), _rows((_BLK, _F))],
    out_shape=[
        jax.ShapeDtypeStruct((_N, 1), jnp.float32),
        jax.ShapeDtypeStruct((_N, _F), jnp.float32),
    ],
)


def _mid_body(u_ref, s_ref, o_ref):
    s = s_ref[...]
    o_ref[...] = -(s * s) * (u_ref[0] + u_ref[1])


_mid = pl.pallas_call(
    _mid_body,
    grid=(_GRID,),
    in_specs=[_rows((2, _BLK, _F)), _rows((_BLK, 1))],
    out_specs=_rows((_BLK, _F)),
    out_shape=jax.ShapeDtypeStruct((_N, _F), jnp.float32),
)


def _layer_body(h_ref, u1_ref, u2_ref, s_ref, A_ref, W1_ref, W2_ref, b_ref,
                ho_ref, sho_ref):
    s = s_ref[...]
    t1 = s * (u1_ref[0] + u1_ref[1])
    t2 = s * (u2_ref[0] + u2_ref[1])
    o = (jnp.dot(h_ref[...], A_ref[...], preferred_element_type=jnp.float32)
         - jnp.dot(t1, W1_ref[...], preferred_element_type=jnp.float32)
         - 2.0 * jnp.dot(t2, W2_ref[...], preferred_element_type=jnp.float32)
         + b_ref[...])
    o = jnp.clip(o, 0.0, 6.0)
    ho_ref[...] = o
    sho_ref[...] = s * o


_layer = pl.pallas_call(
    _layer_body,
    grid=(_GRID,),
    in_specs=[_rows((_BLK, _F)), _rows((2, _BLK, _F)), _rows((2, _BLK, _F)),
              _rows((_BLK, 1)), _full((_F, _F)), _full((_F, _F)),
              _full((_F, _F)), _full((1, _F))],
    out_specs=[_rows((_BLK, _F)), _rows((_BLK, _F))],
    out_shape=[
        jax.ShapeDtypeStruct((_N, _F), jnp.float32),
        jax.ShapeDtypeStruct((_N, _F), jnp.float32),
    ],
)


def _out3_body(h_ref, u1_ref, u2_ref, s_ref, A_ref, W1_ref, W2_ref, b_ref,
               o_ref):
    s = s_ref[...]
    t1 = s * (u1_ref[0] + u1_ref[1])
    t2 = s * (u2_ref[0] + u2_ref[1])
    o_ref[...] = (jnp.dot(h_ref[...], A_ref[...],
                          preferred_element_type=jnp.float32)
                  - jnp.dot(t1, W1_ref[...],
                            preferred_element_type=jnp.float32)
                  - 2.0 * jnp.dot(t2, W2_ref[...],
                                  preferred_element_type=jnp.float32)
                  + b_ref[...])


_out3 = pl.pallas_call(
    _out3_body,
    grid=(_GRID,),
    in_specs=[_rows((_BLK, _F)), _rows((2, _BLK, _F)), _rows((2, _BLK, _F)),
              _rows((_BLK, 1)), _full((_F, _C)), _full((_F, _C)),
              _full((_F, _C)), _full((1, _C))],
    out_specs=_rows((_BLK, _C)),
    out_shape=jax.ShapeDtypeStruct((_N, _C), jnp.float32),
)


def kernel(x, edge_index, W1, b1, W2, b2, W3, b3):
    row = edge_index[0].reshape(_NT, _NSC * _SCH, _CH)
    col = edge_index[1].reshape(_NT, _NSC * _SCH, _CH)
    row2 = edge_index[0].reshape(_NT, _ECT)
    z128 = jnp.zeros((_CH, _F), jnp.float32)
    rix = jnp.arange(_DR, dtype=jnp.int32)

    degp = _deg(row2, rix).reshape(2, _NP, 1)
    s, sh = _pre(degp, x)

    h = x
    for W, b in ((W1, b1), (W2, b2)):
        u1 = _prop128(sh, row, col, z128)
        sT = _mid(u1, s)
        u2 = _prop128(sT, row, col, z128)
        h, sh = _layer(h, u1, u2, s, W[0] - W[2], W[1], W[2],
                       b.reshape(1, -1))

    u1 = _prop128(sh, row, col, z128)
    sT = _mid(u1, s)
    u2 = _prop128(sT, row, col, z128)
    out = _out3(h, u1, u2, s, W3[0] - W3[2], W3[1], W3[2],
                b3.reshape(1, -1))
    return (out, edge_index)


# speedup vs baseline: 1.1424x; 1.1424x over previous
import jax
import jax.numpy as jnp
from jax import lax
from jax.experimental import pallas as pl
from jax.experimental.pallas import tpu as pltpu
from jax.experimental.pallas import tpu_sc as plsc

_N = 10000
_NP = 10240
_E = 320000
_F = 128
_C = 40
_NT = 32
_CH = 80
_NCH = 125
_ECT = 10000
_RPT = _NP // 16
_RB = 80
_NB = 3


def _sc_mesh():
    return plsc.VectorSubcoreMesh(core_axis_name="c", subcore_axis_name="s")


def _make_prop(D):
    def body(g_hbm, pk_hbm, z_hbm, out_hbm, acc, pkv,
             ri0, ri1, ri2, ci0, ci1, ci2, buf0, buf1, buf2,
             gs0, gs1, gs2, ss0, ss1, ss2):
        cid = lax.axis_index("c")
        sid = lax.axis_index("s")
        wid = sid * 2 + cid
        ris = (ri0, ri1, ri2)
        cis = (ci0, ci1, ci2)
        bufs = (buf0, buf1, buf2)
        gss = (gs0, gs1, gs2)
        sss = (ss0, ss1, ss2)

        def unpack(c, j):
            for k in range(_CH // 16):
                pk = pkv[c, pl.ds(k * 16, 16)]
                ris[j][pl.ds(k * 16, 16)] = pk >> 14
                cis[j][pl.ds(k * 16, 16)] = pk & 16383

        def issue_g(j):
            pltpu.async_copy(g_hbm.at[ris[j]], bufs[j], gss[j])

        def wait_g(j):
            pltpu.make_async_copy(g_hbm.at[ris[j]], bufs[j], gss[j]).wait()

        def issue_s(j):
            pltpu.async_copy(bufs[j], acc.at[cis[j]], sss[j], add=True)

        def wait_s(j):
            pltpu.make_async_copy(bufs[j], acc.at[cis[j]], sss[j]).wait()

        pltpu.async_copy(pk_hbm.at[wid], pkv, gs0)
        pltpu.sync_copy(z_hbm, buf0)
        base = sid * _RPT
        for k in range(_RPT // _RB):
            pltpu.sync_copy(buf0, acc.at[pl.ds(base + k * _RB, _RB), :])
        pltpu.make_async_copy(pk_hbm.at[wid], pkv, gs0).wait()
        plsc.subcore_barrier()

        unpack(0, 0)
        issue_g(0)
        unpack(1, 1)
        issue_g(1)
        wait_g(0)
        issue_s(0)
        unpack(2, 2)
        issue_g(2)
        wait_g(1)
        issue_s(1)
        wait_s(0)
        unpack(3, 0)
        issue_g(0)

        def step(i, carry):
            c0 = 3 * i + 2
            for jj in range(3):
                c = c0 + jj
                j = (2 + jj) % 3
                j1 = (1 + jj) % 3
                wait_g(j)
                issue_s(j)
                wait_s(j1)
                unpack(c + 2, j1)
                issue_g(j1)
            return carry

        lax.fori_loop(0, (_NCH - 5) // 3, step, 0)
        wait_g(2)
        issue_s(2)
        wait_s(1)
        unpack(_NCH - 1, 1)
        issue_g(1)
        wait_g(0)
        issue_s(0)
        wait_s(2)
        wait_g(1)
        issue_s(1)
        wait_s(0)
        wait_s(1)

        plsc.subcore_barrier()
        for k in range(_RPT // _RB):
            r0 = base + k * _RB
            pltpu.sync_copy(acc.at[pl.ds(r0, _RB), :], buf0)
            pltpu.sync_copy(buf0, out_hbm.at[cid, pl.ds(r0, _RB), :])

    return pl.kernel(
        body,
        out_type=jax.ShapeDtypeStruct((2, _NP, D), jnp.float32),
        mesh=_sc_mesh(),
        compiler_params=pltpu.CompilerParams(
            disable_bounds_checks=True, disable_semaphore_checks=True),
        scratch_types=[
            pltpu.VMEM_SHARED((_NP, D), jnp.float32),
            pltpu.VMEM((_NCH, _CH), jnp.int32),
            pltpu.VMEM((_CH,), jnp.int32),
            pltpu.VMEM((_CH,), jnp.int32),
            pltpu.VMEM((_CH,), jnp.int32),
            pltpu.VMEM((_CH,), jnp.int32),
            pltpu.VMEM((_CH,), jnp.int32),
            pltpu.VMEM((_CH,), jnp.int32),
            pltpu.VMEM((_CH, D), jnp.float32),
            pltpu.VMEM((_CH, D), jnp.float32),
            pltpu.VMEM((_CH, D), jnp.float32),
            pltpu.SemaphoreType.DMA,
            pltpu.SemaphoreType.DMA,
            pltpu.SemaphoreType.DMA,
            pltpu.SemaphoreType.DMA,
            pltpu.SemaphoreType.DMA,
            pltpu.SemaphoreType.DMA,
        ],
    )


_DR = _NP // _F


def _deg_body(row_hbm, rix_hbm, out_hbm, acc2, degv, rowv, rix, sem0):
    cid = lax.axis_index("c")
    sid = lax.axis_index("s")
    wid = sid * 2 + cid

    ridma = pltpu.async_copy(row_hbm.at[wid], rowv, sem0)
    pltpu.sync_copy(rix_hbm, rix)
    z16 = jnp.zeros((16,), jnp.float32)

    def zstep(r, carry):
        for j in range(_F // 16):
            degv[r, pl.ds(j * 16, 16)] = z16
        return carry

    lax.fori_loop(0, _DR, zstep, 0)
    @pl.when(sid < _DR // 8)
    def _():
        pltpu.sync_copy(degv.at[pl.ds(sid * 8, 8), :],
                        acc2.at[pl.ds(sid * 8, 8), :])

    ridma.wait()
    plsc.subcore_barrier()

    ones16 = jnp.ones((16,), jnp.float32)

    def step(c, carry):
        idx = rowv[pl.ds(c * 16, 16)]
        plsc.addupdate_scatter(degv, [idx >> 7, idx & 127], ones16)
        return carry

    lax.fori_loop(0, _ECT // 16, step, 0)
    pltpu.sync_copy(degv, acc2.at[rix], add=True)
    plsc.subcore_barrier()

    @pl.when(sid < _DR // 8)
    def _():
        pltpu.sync_copy(acc2.at[pl.ds(sid * 8, 8), :],
                        degv.at[pl.ds(0, 8), :])
        pltpu.sync_copy(degv.at[pl.ds(0, 8), :],
                        out_hbm.at[cid, pl.ds(sid * 8, 8), :])


def _make_deg():
    return pl.kernel(
        _deg_body,
        out_type=jax.ShapeDtypeStruct((2, _DR, _F), jnp.float32),
        mesh=_sc_mesh(),
        compiler_params=pltpu.CompilerParams(
            needs_layout_passes=False,
            disable_bounds_checks=True, disable_semaphore_checks=True),
        scratch_types=[
            pltpu.VMEM_SHARED((_DR, _F), jnp.float32),
            pltpu.VMEM((_DR, _F), jnp.float32),
            pltpu.VMEM((_ECT,), jnp.int32),
            pltpu.VMEM((_DR,), jnp.int32),
            pltpu.SemaphoreType.DMA,
        ],
    )


_prop128 = _make_prop(_F)
_deg = _make_deg()


_BLK = 2000
_GRID = _N // _BLK


def _full(shape):
    nd = len(shape)
    return pl.BlockSpec(shape, lambda i, _nd=nd: (0,) * _nd)


def _rows(shape):
    if len(shape) == 3:
        return pl.BlockSpec(shape, lambda i: (0, i, 0))
    return pl.BlockSpec(shape, lambda i: (i, 0))


def _pre_body(deg_ref, x_ref, s_ref, sh_ref):
    d = deg_ref[0] + deg_ref[1]
    s = jnp.where(d > 0.0, lax.rsqrt(d), 0.0)
    s_ref[...] = s
    sh_ref[...] = x_ref[...] * s


_pre = pl.pallas_call(
    _pre_body,
    grid=(_GRID,),
    in_specs=[_rows((2, _BLK, 1)), _rows((_BLK, _F))],
    out_specs=[_rows((_BLK, 1)), _rows((_BLK, _F))],
    out_shape=[
        jax.ShapeDtypeStruct((_N, 1), jnp.float32),
        jax.ShapeDtypeStruct((_N, _F), jnp.float32),
    ],
)


def _mid_body(u_ref, s_ref, o_ref):
    s = s_ref[...]
    o_ref[...] = -(s * s) * (u_ref[0] + u_ref[1])


_mid = pl.pallas_call(
    _mid_body,
    grid=(_GRID,),
    in_specs=[_rows((2, _BLK, _F)), _rows((_BLK, 1))],
    out_specs=_rows((_BLK, _F)),
    out_shape=jax.ShapeDtypeStruct((_N, _F), jnp.float32),
)


def _layer_body(h_ref, u1_ref, u2_ref, s_ref, A_ref, W1_ref, W2_ref, b_ref,
                ho_ref, sho_ref):
    s = s_ref[...]
    t1 = s * (u1_ref[0] + u1_ref[1])
    t2 = s * (u2_ref[0] + u2_ref[1])
    o = (jnp.dot(h_ref[...], A_ref[...], preferred_element_type=jnp.float32)
         - jnp.dot(t1, W1_ref[...], preferred_element_type=jnp.float32)
         - 2.0 * jnp.dot(t2, W2_ref[...], preferred_element_type=jnp.float32)
         + b_ref[...])
    o = jnp.clip(o, 0.0, 6.0)
    ho_ref[...] = o
    sho_ref[...] = s * o


_layer = pl.pallas_call(
    _layer_body,
    grid=(_GRID,),
    in_specs=[_rows((_BLK, _F)), _rows((2, _BLK, _F)), _rows((2, _BLK, _F)),
              _rows((_BLK, 1)), _full((_F, _F)), _full((_F, _F)),
              _full((_F, _F)), _full((1, _F))],
    out_specs=[_rows((_BLK, _F)), _rows((_BLK, _F))],
    out_shape=[
        jax.ShapeDtypeStruct((_N, _F), jnp.float32),
        jax.ShapeDtypeStruct((_N, _F), jnp.float32),
    ],
)


def _out3_body(h_ref, u1_ref, u2_ref, s_ref, A_ref, W1_ref, W2_ref, b_ref,
               o_ref):
    s = s_ref[...]
    t1 = s * (u1_ref[0] + u1_ref[1])
    t2 = s * (u2_ref[0] + u2_ref[1])
    o_ref[...] = (jnp.dot(h_ref[...], A_ref[...],
                          preferred_element_type=jnp.float32)
                  - jnp.dot(t1, W1_ref[...],
                            preferred_element_type=jnp.float32)
                  - 2.0 * jnp.dot(t2, W2_ref[...],
                                  preferred_element_type=jnp.float32)
                  + b_ref[...])


_out3 = pl.pallas_call(
    _out3_body,
    grid=(_GRID,),
    in_specs=[_rows((_BLK, _F)), _rows((2, _BLK, _F)), _rows((2, _BLK, _F)),
              _rows((_BLK, 1)), _full((_F, _C)), _full((_F, _C)),
              _full((_F, _C)), _full((1, _C))],
    out_specs=_rows((_BLK, _C)),
    out_shape=jax.ShapeDtypeStruct((_N, _C), jnp.float32),
)


def kernel(x, edge_index, W1, b1, W2, b2, W3, b3):
    pk = ((edge_index[0] << 14) | edge_index[1]).reshape(_NT, _NCH, _CH)
    row2 = edge_index[0].reshape(_NT, _ECT)
    z128 = jnp.zeros((_CH, _F), jnp.float32)
    rix = jnp.arange(_DR, dtype=jnp.int32)

    degp = _deg(row2, rix).reshape(2, _NP, 1)
    s, sh = _pre(degp, x)

    h = x
    for W, b in ((W1, b1), (W2, b2)):
        u1 = _prop128(sh, pk, z128)
        sT = _mid(u1, s)
        u2 = _prop128(sT, pk, z128)
        h, sh = _layer(h, u1, u2, s, W[0] - W[2], W[1], W[2],
                       b.reshape(1, -1))

    u1 = _prop128(sh, pk, z128)
    sT = _mid(u1, s)
    u2 = _prop128(sT, pk, z128)
    out = _out3(h, u1, u2, s, W3[0] - W3[2], W3[1], W3[2],
                b3.reshape(1, -1))
    return (out, edge_index)


# speedup vs baseline: 1.1743x; 1.0279x over previous
import jax
import jax.numpy as jnp
from jax import lax
from jax.experimental import pallas as pl
from jax.experimental.pallas import tpu as pltpu
from jax.experimental.pallas import tpu_sc as plsc

_N = 10000
_NP = 10240
_E = 320000
_F = 128
_C = 40
_NT = 32
_CH = 80
_NCH = 125
_ECT = 10000
_RPT = _NP // 16
_RB = 80
_NB = 3


def _sc_mesh():
    return plsc.VectorSubcoreMesh(core_axis_name="c", subcore_axis_name="s")


def _make_prop(D):
    def body(g_hbm, pk_hbm, z_hbm, out_hbm, acc, pkv,
             ri0, ri1, ri2, ci0, ci1, ci2, buf0, buf1, buf2,
             gs0, gs1, gs2, ss0, ss1, ss2):
        cid = lax.axis_index("c")
        sid = lax.axis_index("s")
        wid = sid * 2 + cid
        ris = (ri0, ri1, ri2)
        cis = (ci0, ci1, ci2)
        bufs = (buf0, buf1, buf2)
        gss = (gs0, gs1, gs2)
        sss = (ss0, ss1, ss2)

        def unpack(c, j):
            for k in range(_CH // 16):
                pk = pkv[c, pl.ds(k * 16, 16)]
                ris[j][pl.ds(k * 16, 16)] = pk >> 14
                cis[j][pl.ds(k * 16, 16)] = pk & 16383

        def issue_g(j):
            pltpu.async_copy(g_hbm.at[ris[j]], bufs[j], gss[j])

        def wait_g(j):
            pltpu.make_async_copy(g_hbm.at[ris[j]], bufs[j], gss[j]).wait()

        def issue_s(j):
            pltpu.async_copy(bufs[j], acc.at[cis[j]], sss[j], add=True)

        def wait_s(j):
            pltpu.make_async_copy(bufs[j], acc.at[cis[j]], sss[j]).wait()

        pltpu.async_copy(pk_hbm.at[wid], pkv, gs0)
        pltpu.sync_copy(z_hbm, buf0)
        base = sid * _RPT
        for k in range(_RPT // _RB):
            pltpu.sync_copy(buf0, acc.at[pl.ds(base + k * _RB, _RB), :])
        pltpu.make_async_copy(pk_hbm.at[wid], pkv, gs0).wait()
        plsc.subcore_barrier()

        unpack(0, 0)
        issue_g(0)
        unpack(1, 1)
        issue_g(1)
        wait_g(0)
        issue_s(0)
        unpack(2, 2)
        issue_g(2)
        wait_g(1)
        issue_s(1)
        wait_s(0)
        unpack(3, 0)
        issue_g(0)

        def step(i, carry):
            c0 = 3 * i + 2
            for jj in range(3):
                c = c0 + jj
                j = (2 + jj) % 3
                j1 = (1 + jj) % 3
                wait_g(j)
                issue_s(j)
                wait_s(j1)
                unpack(c + 2, j1)
                issue_g(j1)
            return carry

        lax.fori_loop(0, (_NCH - 5) // 3, step, 0)
        wait_g(2)
        issue_s(2)
        wait_s(1)
        unpack(_NCH - 1, 1)
        issue_g(1)
        wait_g(0)
        issue_s(0)
        wait_s(2)
        wait_g(1)
        issue_s(1)
        wait_s(0)
        wait_s(1)

        plsc.subcore_barrier()
        for k in range(_RPT // _RB):
            r0 = base + k * _RB
            pltpu.sync_copy(acc.at[pl.ds(r0, _RB), :], buf0)
            pltpu.sync_copy(buf0, out_hbm.at[cid, pl.ds(r0, _RB), :])

    return pl.kernel(
        body,
        out_type=jax.ShapeDtypeStruct((2, _NP, D), jnp.float32),
        mesh=_sc_mesh(),
        compiler_params=pltpu.CompilerParams(
            disable_bounds_checks=True, disable_semaphore_checks=True,
            use_tc_tiling_on_sc=(D % 128 == 0)),
        scratch_types=[
            pltpu.VMEM_SHARED((_NP, D), jnp.float32),
            pltpu.VMEM((_NCH, _CH), jnp.int32),
            pltpu.VMEM((_CH,), jnp.int32),
            pltpu.VMEM((_CH,), jnp.int32),
            pltpu.VMEM((_CH,), jnp.int32),
            pltpu.VMEM((_CH,), jnp.int32),
            pltpu.VMEM((_CH,), jnp.int32),
            pltpu.VMEM((_CH,), jnp.int32),
            pltpu.VMEM((_CH, D), jnp.float32),
            pltpu.VMEM((_CH, D), jnp.float32),
            pltpu.VMEM((_CH, D), jnp.float32),
            pltpu.SemaphoreType.DMA,
            pltpu.SemaphoreType.DMA,
            pltpu.SemaphoreType.DMA,
            pltpu.SemaphoreType.DMA,
            pltpu.SemaphoreType.DMA,
            pltpu.SemaphoreType.DMA,
        ],
    )


_DR = _NP // _F


def _deg_body(row_hbm, rix_hbm, out_hbm, acc2, degv, rowv, rix, sem0):
    cid = lax.axis_index("c")
    sid = lax.axis_index("s")
    wid = sid * 2 + cid

    ridma = pltpu.async_copy(row_hbm.at[wid], rowv, sem0)
    pltpu.sync_copy(rix_hbm, rix)
    z16 = jnp.zeros((16,), jnp.float32)

    def zstep(r, carry):
        for j in range(_F // 16):
            degv[r, pl.ds(j * 16, 16)] = z16
        return carry

    lax.fori_loop(0, _DR, zstep, 0)
    @pl.when(sid < _DR // 8)
    def _():
        pltpu.sync_copy(degv.at[pl.ds(sid * 8, 8), :],
                        acc2.at[pl.ds(sid * 8, 8), :])

    ridma.wait()
    plsc.subcore_barrier()

    ones16 = jnp.ones((16,), jnp.float32)

    def step(c, carry):
        idx = rowv[pl.ds(c * 16, 16)]
        plsc.addupdate_scatter(degv, [idx >> 7, idx & 127], ones16)
        return carry

    lax.fori_loop(0, _ECT // 16, step, 0)
    pltpu.sync_copy(degv, acc2.at[rix], add=True)
    plsc.subcore_barrier()

    @pl.when(sid < _DR // 8)
    def _():
        pltpu.sync_copy(acc2.at[pl.ds(sid * 8, 8), :],
                        degv.at[pl.ds(0, 8), :])
        pltpu.sync_copy(degv.at[pl.ds(0, 8), :],
                        out_hbm.at[cid, pl.ds(sid * 8, 8), :])


def _make_deg():
    return pl.kernel(
        _deg_body,
        out_type=jax.ShapeDtypeStruct((2, _DR, _F), jnp.float32),
        mesh=_sc_mesh(),
        compiler_params=pltpu.CompilerParams(
            needs_layout_passes=False,
            disable_bounds_checks=True, disable_semaphore_checks=True),
        scratch_types=[
            pltpu.VMEM_SHARED((_DR, _F), jnp.float32),
            pltpu.VMEM((_DR, _F), jnp.float32),
            pltpu.VMEM((_ECT,), jnp.int32),
            pltpu.VMEM((_DR,), jnp.int32),
            pltpu.SemaphoreType.DMA,
        ],
    )


_prop128 = _make_prop(_F)
_prop40 = _make_prop(_C)
_deg = _make_deg()


_BLK = 2000
_GRID = _N // _BLK


def _full(shape):
    nd = len(shape)
    return pl.BlockSpec(shape, lambda i, _nd=nd: (0,) * _nd)


def _rows(shape):
    if len(shape) == 3:
        return pl.BlockSpec(shape, lambda i: (0, i, 0))
    return pl.BlockSpec(shape, lambda i: (i, 0))


def _pre_body(deg_ref, x_ref, s_ref, sh_ref):
    d = deg_ref[0] + deg_ref[1]
    s = jnp.where(d > 0.0, lax.rsqrt(d), 0.0)
    s_ref[...] = s
    sh_ref[...] = x_ref[...] * s


_pre = pl.pallas_call(
    _pre_body,
    grid=(_GRID,),
    in_specs=[_rows((2, _BLK, 1)), _rows((_BLK, _F))],
    out_specs=[_rows((_BLK, 1)), _rows((_BLK, _F))],
    out_shape=[
        jax.ShapeDtypeStruct((_N, 1), jnp.float32),
        jax.ShapeDtypeStruct((_N, _F), jnp.float32),
    ],
)


def _mid_body(u_ref, s_ref, o_ref):
    s = s_ref[...]
    o_ref[...] = -(s * s) * (u_ref[0] + u_ref[1])


_mid = pl.pallas_call(
    _mid_body,
    grid=(_GRID,),
    in_specs=[_rows((2, _BLK, _F)), _rows((_BLK, 1))],
    out_specs=_rows((_BLK, _F)),
    out_shape=jax.ShapeDtypeStruct((_N, _F), jnp.float32),
)


def _layer_body(h_ref, u1_ref, u2_ref, s_ref, A_ref, W1_ref, W2_ref, b_ref,
                ho_ref, sho_ref):
    s = s_ref[...]
    t1 = s * (u1_ref[0] + u1_ref[1])
    t2 = s * (u2_ref[0] + u2_ref[1])
    o = (jnp.dot(h_ref[...], A_ref[...], preferred_element_type=jnp.float32)
         - jnp.dot(t1, W1_ref[...], preferred_element_type=jnp.float32)
         - 2.0 * jnp.dot(t2, W2_ref[...], preferred_element_type=jnp.float32)
         + b_ref[...])
    o = jnp.clip(o, 0.0, 6.0)
    ho_ref[...] = o
    sho_ref[...] = s * o


_layer = pl.pallas_call(
    _layer_body,
    grid=(_GRID,),
    in_specs=[_rows((_BLK, _F)), _rows((2, _BLK, _F)), _rows((2, _BLK, _F)),
              _rows((_BLK, 1)), _full((_F, _F)), _full((_F, _F)),
              _full((_F, _F)), _full((1, _F))],
    out_specs=[_rows((_BLK, _F)), _rows((_BLK, _F))],
    out_shape=[
        jax.ShapeDtypeStruct((_N, _F), jnp.float32),
        jax.ShapeDtypeStruct((_N, _F), jnp.float32),
    ],
)


def _mid3_body(u_ref, s_ref, W_ref, o_ref):
    s = s_ref[...]
    t = -(s * s) * (u_ref[0] + u_ref[1])
    o_ref[...] = jnp.dot(t, W_ref[...], preferred_element_type=jnp.float32)


_mid3 = pl.pallas_call(
    _mid3_body,
    grid=(_GRID,),
    in_specs=[_rows((2, _BLK, _F)), _rows((_BLK, 1)), _full((_F, _C))],
    out_specs=_rows((_BLK, _C)),
    out_shape=jax.ShapeDtypeStruct((_N, _C), jnp.float32),
)


def _out3_body(h_ref, u1_ref, u2_ref, s_ref, A_ref, W1_ref, b_ref, o_ref):
    s = s_ref[...]
    t1 = s * (u1_ref[0] + u1_ref[1])
    o_ref[...] = (jnp.dot(h_ref[...], A_ref[...],
                          preferred_element_type=jnp.float32)
                  - jnp.dot(t1, W1_ref[...],
                            preferred_element_type=jnp.float32)
                  - s * (u2_ref[0] + u2_ref[1])
                  + b_ref[...])


_out3 = pl.pallas_call(
    _out3_body,
    grid=(_GRID,),
    in_specs=[_rows((_BLK, _F)), _rows((2, _BLK, _F)), _rows((2, _BLK, _C)),
              _rows((_BLK, 1)), _full((_F, _C)), _full((_F, _C)),
              _full((1, _C))],
    out_specs=_rows((_BLK, _C)),
    out_shape=jax.ShapeDtypeStruct((_N, _C), jnp.float32),
)


def kernel(x, edge_index, W1, b1, W2, b2, W3, b3):
    pk = ((edge_index[0] << 14) | edge_index[1]).reshape(_NT, _NCH, _CH)
    row2 = edge_index[0].reshape(_NT, _ECT)
    z128 = jnp.zeros((_CH, _F), jnp.float32)
    z40 = jnp.zeros((_CH, _C), jnp.float32)
    rix = jnp.arange(_DR, dtype=jnp.int32)

    degp = _deg(row2, rix).reshape(2, _NP, 1)
    s, sh = _pre(degp, x)

    h = x
    for W, b in ((W1, b1), (W2, b2)):
        u1 = _prop128(sh, pk, z128)
        sT = _mid(u1, s)
        u2 = _prop128(sT, pk, z128)
        h, sh = _layer(h, u1, u2, s, W[0] - W[2], W[1], W[2],
                       b.reshape(1, -1))

    u1 = _prop128(sh, pk, z128)
    v = _mid3(u1, s, 2.0 * W3[2])
    u2 = _prop40(v, pk, z40)
    out = _out3(h, u1, u2, s, W3[0] - W3[2], W3[1], b3.reshape(1, -1))
    return (out, edge_index)


# speedup vs baseline: 1.1869x; 1.0108x over previous
import jax
import jax.numpy as jnp
from jax import lax
from jax.experimental import pallas as pl
from jax.experimental.pallas import tpu as pltpu
from jax.experimental.pallas import tpu_sc as plsc

_N = 10000
_NP = 10240
_E = 320000
_F = 128
_C = 40
_NT = 32
_CH = 80
_NCH = 125
_ECT = 10000
_RPT = _NP // 16
_RB = 80
_NB = 3


def _sc_mesh():
    return plsc.VectorSubcoreMesh(core_axis_name="c", subcore_axis_name="s")


def _make_prop(D):
    def body(g_hbm, pk_hbm, z_hbm, out_hbm, acc, pkv,
             ri0, ri1, ri2, ci0, ci1, ci2, buf0, buf1, buf2,
             gs0, gs1, gs2, ss0, ss1, ss2):
        cid = lax.axis_index("c")
        sid = lax.axis_index("s")
        wid = sid * 2 + cid
        ris = (ri0, ri1, ri2)
        cis = (ci0, ci1, ci2)
        bufs = (buf0, buf1, buf2)
        gss = (gs0, gs1, gs2)
        sss = (ss0, ss1, ss2)

        def unpack(c, j):
            for k in range(_CH // 16):
                pk = pkv[c, pl.ds(k * 16, 16)]
                ris[j][pl.ds(k * 16, 16)] = pk >> 14
                cis[j][pl.ds(k * 16, 16)] = pk & 16383

        def issue_g(j):
            pltpu.async_copy(g_hbm.at[ris[j]], bufs[j], gss[j])

        def wait_g(j):
            pltpu.make_async_copy(g_hbm.at[ris[j]], bufs[j], gss[j]).wait()

        def issue_s(j):
            pltpu.async_copy(bufs[j], acc.at[cis[j]], sss[j], add=True)

        def wait_s(j):
            pltpu.make_async_copy(bufs[j], acc.at[cis[j]], sss[j]).wait()

        pltpu.async_copy(pk_hbm.at[wid], pkv, gs0)
        pltpu.sync_copy(z_hbm, buf0)
        base = sid * _RPT
        for k in range(_RPT // _RB):
            pltpu.async_copy(buf0, acc.at[pl.ds(base + k * _RB, _RB), :], ss0)
        for k in range(_RPT // _RB):
            pltpu.make_async_copy(
                buf0, acc.at[pl.ds(base + k * _RB, _RB), :], ss0).wait()
        pltpu.make_async_copy(pk_hbm.at[wid], pkv, gs0).wait()
        plsc.subcore_barrier()

        unpack(0, 0)
        issue_g(0)
        unpack(1, 1)
        issue_g(1)
        wait_g(0)
        issue_s(0)
        unpack(2, 2)
        issue_g(2)
        wait_g(1)
        issue_s(1)
        wait_s(0)
        unpack(3, 0)
        issue_g(0)

        def step(i, carry):
            c0 = 3 * i + 2
            for jj in range(3):
                c = c0 + jj
                j = (2 + jj) % 3
                j1 = (1 + jj) % 3
                wait_g(j)
                issue_s(j)
                wait_s(j1)
                unpack(c + 2, j1)
                issue_g(j1)
            return carry

        lax.fori_loop(0, (_NCH - 5) // 3, step, 0)
        wait_g(2)
        issue_s(2)
        wait_s(1)
        unpack(_NCH - 1, 1)
        issue_g(1)
        wait_g(0)
        issue_s(0)
        wait_s(2)
        wait_g(1)
        issue_s(1)
        wait_s(0)
        wait_s(1)

        plsc.subcore_barrier()
        pltpu.sync_copy(acc.at[pl.ds(base, _RPT), :],
                        out_hbm.at[cid, pl.ds(base, _RPT), :])

    return pl.kernel(
        body,
        out_type=jax.ShapeDtypeStruct((2, _NP, D), jnp.float32),
        mesh=_sc_mesh(),
        compiler_params=pltpu.CompilerParams(
            disable_bounds_checks=True, disable_semaphore_checks=True,
            use_tc_tiling_on_sc=(D % 128 == 0)),
        scratch_types=[
            pltpu.VMEM_SHARED((_NP, D), jnp.float32),
            pltpu.VMEM((_NCH, _CH), jnp.int32),
            pltpu.VMEM((_CH,), jnp.int32),
            pltpu.VMEM((_CH,), jnp.int32),
            pltpu.VMEM((_CH,), jnp.int32),
            pltpu.VMEM((_CH,), jnp.int32),
            pltpu.VMEM((_CH,), jnp.int32),
            pltpu.VMEM((_CH,), jnp.int32),
            pltpu.VMEM((_CH, D), jnp.float32),
            pltpu.VMEM((_CH, D), jnp.float32),
            pltpu.VMEM((_CH, D), jnp.float32),
            pltpu.SemaphoreType.DMA,
            pltpu.SemaphoreType.DMA,
            pltpu.SemaphoreType.DMA,
            pltpu.SemaphoreType.DMA,
            pltpu.SemaphoreType.DMA,
            pltpu.SemaphoreType.DMA,
        ],
    )


_DR = _NP // _F


def _deg_body(row_hbm, rix_hbm, out_hbm, acc2, degv, rowv, rix, sem0):
    cid = lax.axis_index("c")
    sid = lax.axis_index("s")
    wid = sid * 2 + cid

    ridma = pltpu.async_copy(row_hbm.at[wid], rowv, sem0)
    pltpu.sync_copy(rix_hbm, rix)
    z16 = jnp.zeros((16,), jnp.float32)

    def zstep(r, carry):
        for j in range(_F // 16):
            degv[r, pl.ds(j * 16, 16)] = z16
        return carry

    lax.fori_loop(0, _DR, zstep, 0)
    @pl.when(sid < _DR // 8)
    def _():
        pltpu.sync_copy(degv.at[pl.ds(sid * 8, 8), :],
                        acc2.at[pl.ds(sid * 8, 8), :])

    ridma.wait()
    plsc.subcore_barrier()

    ones16 = jnp.ones((16,), jnp.float32)

    def step(c, carry):
        idx = rowv[pl.ds(c * 16, 16)]
        plsc.addupdate_scatter(degv, [idx >> 7, idx & 127], ones16)
        return carry

    lax.fori_loop(0, _ECT // 16, step, 0)
    pltpu.sync_copy(degv, acc2.at[rix], add=True)
    plsc.subcore_barrier()

    @pl.when(sid < _DR // 8)
    def _():
        pltpu.sync_copy(acc2.at[pl.ds(sid * 8, 8), :],
                        degv.at[pl.ds(0, 8), :])
        pltpu.sync_copy(degv.at[pl.ds(0, 8), :],
                        out_hbm.at[cid, pl.ds(sid * 8, 8), :])


def _make_deg():
    return pl.kernel(
        _deg_body,
        out_type=jax.ShapeDtypeStruct((2, _DR, _F), jnp.float32),
        mesh=_sc_mesh(),
        compiler_params=pltpu.CompilerParams(
            needs_layout_passes=False,
            disable_bounds_checks=True, disable_semaphore_checks=True),
        scratch_types=[
            pltpu.VMEM_SHARED((_DR, _F), jnp.float32),
            pltpu.VMEM((_DR, _F), jnp.float32),
            pltpu.VMEM((_ECT,), jnp.int32),
            pltpu.VMEM((_DR,), jnp.int32),
            pltpu.SemaphoreType.DMA,
        ],
    )


_prop128 = _make_prop(_F)
_prop40 = _make_prop(_C)
_deg = _make_deg()


_BLK = 2000
_GRID = _N // _BLK


def _full(shape):
    nd = len(shape)
    return pl.BlockSpec(shape, lambda i, _nd=nd: (0,) * _nd)


def _rows(shape):
    if len(shape) == 3:
        return pl.BlockSpec(shape, lambda i: (0, i, 0))
    return pl.BlockSpec(shape, lambda i: (i, 0))


def _pre_body(deg_ref, x_ref, s_ref, sh_ref):
    d = deg_ref[0] + deg_ref[1]
    s = jnp.where(d > 0.0, lax.rsqrt(d), 0.0)
    s_ref[...] = s
    sh_ref[...] = x_ref[...] * s


_pre = pl.pallas_call(
    _pre_body,
    grid=(_GRID,),
    in_specs=[_rows((2, _BLK, 1)), _rows((_BLK, _F))],
    out_specs=[_rows((_BLK, 1)), _rows((_BLK, _F))],
    out_shape=[
        jax.ShapeDtypeStruct((_N, 1), jnp.float32),
        jax.ShapeDtypeStruct((_N, _F), jnp.float32),
    ],
)


def _mid_body(u_ref, s_ref, o_ref):
    s = s_ref[...]
    o_ref[...] = -(s * s) * (u_ref[0] + u_ref[1])


_mid = pl.pallas_call(
    _mid_body,
    grid=(_GRID,),
    in_specs=[_rows((2, _BLK, _F)), _rows((_BLK, 1))],
    out_specs=_rows((_BLK, _F)),
    out_shape=jax.ShapeDtypeStruct((_N, _F), jnp.float32),
)


def _layer_body(h_ref, u1_ref, u2_ref, s_ref, A_ref, W1_ref, W2_ref, b_ref,
                ho_ref, sho_ref):
    s = s_ref[...]
    t1 = s * (u1_ref[0] + u1_ref[1])
    t2 = s * (u2_ref[0] + u2_ref[1])
    o = (jnp.dot(h_ref[...], A_ref[...], preferred_element_type=jnp.float32)
         - jnp.dot(t1, W1_ref[...], preferred_element_type=jnp.float32)
         - 2.0 * jnp.dot(t2, W2_ref[...], preferred_element_type=jnp.float32)
         + b_ref[...])
    o = jnp.clip(o, 0.0, 6.0)
    ho_ref[...] = o
    sho_ref[...] = s * o


_layer = pl.pallas_call(
    _layer_body,
    grid=(_GRID,),
    in_specs=[_rows((_BLK, _F)), _rows((2, _BLK, _F)), _rows((2, _BLK, _F)),
              _rows((_BLK, 1)), _full((_F, _F)), _full((_F, _F)),
              _full((_F, _F)), _full((1, _F))],
    out_specs=[_rows((_BLK, _F)), _rows((_BLK, _F))],
    out_shape=[
        jax.ShapeDtypeStruct((_N, _F), jnp.float32),
        jax.ShapeDtypeStruct((_N, _F), jnp.float32),
    ],
)


def _mid3_body(u_ref, s_ref, W_ref, o_ref):
    s = s_ref[...]
    t = -(s * s) * (u_ref[0] + u_ref[1])
    o_ref[...] = jnp.dot(t, W_ref[...], preferred_element_type=jnp.float32)


_mid3 = pl.pallas_call(
    _mid3_body,
    grid=(_GRID,),
    in_specs=[_rows((2, _BLK, _F)), _rows((_BLK, 1)), _full((_F, _C))],
    out_specs=_rows((_BLK, _C)),
    out_shape=jax.ShapeDtypeStruct((_N, _C), jnp.float32),
)


def _out3_body(h_ref, u1_ref, u2_ref, s_ref, A_ref, W1_ref, b_ref, o_ref):
    s = s_ref[...]
    t1 = s * (u1_ref[0] + u1_ref[1])
    o_ref[...] = (jnp.dot(h_ref[...], A_ref[...],
                          preferred_element_type=jnp.float32)
                  - jnp.dot(t1, W1_ref[...],
                            preferred_element_type=jnp.float32)
                  - s * (u2_ref[0] + u2_ref[1])
                  + b_ref[...])


_out3 = pl.pallas_call(
    _out3_body,
    grid=(_GRID,),
    in_specs=[_rows((_BLK, _F)), _rows((2, _BLK, _F)), _rows((2, _BLK, _C)),
              _rows((_BLK, 1)), _full((_F, _C)), _full((_F, _C)),
              _full((1, _C))],
    out_specs=_rows((_BLK, _C)),
    out_shape=jax.ShapeDtypeStruct((_N, _C), jnp.float32),
)


def kernel(x, edge_index, W1, b1, W2, b2, W3, b3):
    pk = ((edge_index[0] << 14) | edge_index[1]).reshape(_NT, _NCH, _CH)
    row2 = edge_index[0].reshape(_NT, _ECT)
    z128 = jnp.zeros((_CH, _F), jnp.float32)
    z40 = jnp.zeros((_CH, _C), jnp.float32)
    rix = jnp.arange(_DR, dtype=jnp.int32)

    degp = _deg(row2, rix).reshape(2, _NP, 1)
    s, sh = _pre(degp, x)

    h = x
    for W, b in ((W1, b1), (W2, b2)):
        u1 = _prop128(sh, pk, z128)
        sT = _mid(u1, s)
        u2 = _prop128(sT, pk, z128)
        h, sh = _layer(h, u1, u2, s, W[0] - W[2], W[1], W[2],
                       b.reshape(1, -1))

    u1 = _prop128(sh, pk, z128)
    v = _mid3(u1, s, 2.0 * W3[2])
    u2 = _prop40(v, pk, z40)
    out = _out3(h, u1, u2, s, W3[0] - W3[2], W3[1], b3.reshape(1, -1))
    return (out, edge_index)
